# baseline (speedup 1.0000x reference)
import functools

import jax
import jax.numpy as jnp
from jax import lax
from jax.experimental import pallas as pl
from jax.experimental.pallas import tpu as pltpu
from jax.experimental.pallas import tpu_sc as plsc

K = 32
NC = 2
NS = 16
NW = NC * NS
IDX_CHUNK = 128
TC_BLK = 2048


@functools.lru_cache(maxsize=None)
def _build_gather(B):
    BPW = B // NW
    NCHUNK = BPW // IDX_CHUNK

    mesh = plsc.VectorSubcoreMesh(core_axis_name="c", subcore_axis_name="s")

    @functools.partial(
        pl.kernel,
        mesh=mesh,
        compiler_params=pltpu.CompilerParams(use_tc_tiling_on_sc=False),
        out_type=(
            jax.ShapeDtypeStruct((B, K), jnp.float32),
            jax.ShapeDtypeStruct((B, K), jnp.float32),
        ),
        scratch_types=[
            pltpu.VMEM((NCHUNK, IDX_CHUNK), jnp.int32),
            pltpu.VMEM((NCHUNK, IDX_CHUNK), jnp.int32),
            pltpu.VMEM((BPW, K), jnp.float32),
            pltpu.VMEM((BPW, K), jnp.float32),
            pltpu.SemaphoreType.DMA,
            pltpu.SemaphoreType.DMA,
        ],
    )
    def sc_gather(uidx_hbm, iidx_hbm, ut_hbm, it_hbm, ubuf_hbm, ibuf_hbm,
                  uidx_v, iidx_v, urows, irows, sem_g, sem_w):
        wid = lax.axis_index("s") * NC + lax.axis_index("c")
        base = wid * BPW
        pltpu.sync_copy(uidx_hbm.at[wid], uidx_v)
        pltpu.sync_copy(iidx_hbm.at[wid], iidx_v)

        gathers = []
        for j in range(NCHUNK):
            gathers.append(pltpu.async_copy(
                ut_hbm.at[uidx_v.at[j]],
                urows.at[pl.ds(j * IDX_CHUNK, IDX_CHUNK)], sem_g))
        for j in range(NCHUNK):
            gathers.append(pltpu.async_copy(
                it_hbm.at[iidx_v.at[j]],
                irows.at[pl.ds(j * IDX_CHUNK, IDX_CHUNK)], sem_g))
        for g in gathers[:NCHUNK]:
            g.wait()
        wb_u = pltpu.async_copy(urows, ubuf_hbm.at[pl.ds(base, BPW)], sem_w)
        for g in gathers[NCHUNK:]:
            g.wait()
        wb_i = pltpu.async_copy(irows, ibuf_hbm.at[pl.ds(base, BPW)], sem_w)
        wb_u.wait()
        wb_i.wait()

    return sc_gather


def _tc_dense(u_ref, i_ref, w_ref, b_ref, o_ref):
    wu = w_ref[0:K, :]
    wi = w_ref[K:2 * K, :]
    s = jnp.dot(u_ref[...], wu, preferred_element_type=jnp.float32)
    s = s + jnp.dot(i_ref[...], wi, preferred_element_type=jnp.float32)
    o_ref[...] = jnp.maximum(s + b_ref[0, 0], 0.0)


@functools.lru_cache(maxsize=None)
def _build_dense(B):
    nblk = B // TC_BLK
    return pl.pallas_call(
        _tc_dense,
        grid=(nblk,),
        in_specs=[
            pl.BlockSpec((TC_BLK, K), lambda i: (i, 0)),
            pl.BlockSpec((TC_BLK, K), lambda i: (i, 0)),
            pl.BlockSpec((2 * K, 1), lambda i: (0, 0)),
            pl.BlockSpec((1, 1), lambda i: (0, 0)),
        ],
        out_specs=pl.BlockSpec((TC_BLK, 1), lambda i: (i, 0)),
        out_shape=jax.ShapeDtypeStruct((B, 1), jnp.float32),
    )


def kernel(user_ids, item_ids, user_table, item_table, dense_w, dense_b):
    B = user_ids.shape[0]
    uidx = user_ids.astype(jnp.int32).reshape(NW, -1, IDX_CHUNK)
    iidx = item_ids.astype(jnp.int32).reshape(NW, -1, IDX_CHUNK)
    ubuf, ibuf = _build_gather(B)(uidx, iidx, user_table, item_table)
    return _build_dense(B)(ubuf, ibuf, dense_w, dense_b.reshape(1, 1))
